# Initial kernel scaffold; baseline (speedup 1.0000x reference)
import jax
import jax.numpy as jnp
from jax.experimental import pallas as pl


def kernel(inputs):
    raise NotImplementedError("write your pallas kernel here")


# speedup vs baseline: 22.9056x; 22.9056x over previous
import functools

import jax
import jax.numpy as jnp
from jax import lax
from jax.experimental import pallas as pl
from jax.experimental.pallas import tpu as pltpu
from jax.experimental.pallas import tpu_sc as plsc

B = 4
L = 8192
C = 768
K = 8
LANES = 16
NCG = C // LANES
NTASK = B * NCG
NW = 32
TPW = NTASK // NW
LC = 1024
NCHUNK = L // LC


@functools.partial(
    pl.kernel,
    mesh=plsc.VectorSubcoreMesh(core_axis_name="c", subcore_axis_name="s"),
    out_type=jax.ShapeDtypeStruct((B, K, C), jnp.float32),
    scratch_types=[
        pltpu.VMEM((LC, LANES), jnp.float32),
        pltpu.VMEM((LC, LANES), jnp.float32),
        pltpu.VMEM((K, LANES), jnp.float32),
        pltpu.SemaphoreType.DMA,
        pltpu.SemaphoreType.DMA,
    ],
    compiler_params=pltpu.CompilerParams(use_tc_tiling_on_sc=False),
)
def _topk_sc(x_hbm, out_hbm, buf0, buf1, outb, sem0, sem1):
    wid = lax.axis_index("s") * 2 + lax.axis_index("c")
    bufs = (buf0, buf1)
    sems = (sem0, sem1)

    for t in range(TPW):
        g = t * NW + wid
        b = g // NCG
        cg = g - b * NCG
        c0 = cg * LANES

        def start(chunk, slot, b=b, c0=c0):
            return pltpu.async_copy(
                x_hbm.at[b, pl.ds(chunk * LC, LC), pl.ds(c0, LANES)],
                bufs[slot],
                sems[slot],
            )

        copies = [None, None]
        copies[0] = start(0, 0)
        neg_inf = jnp.full((LANES,), -jnp.inf, dtype=jnp.float32)
        V = tuple(neg_inf for _ in range(K))

        for chunk in range(NCHUNK):
            slot = chunk % 2
            copies[slot].wait()
            if chunk + 1 < NCHUNK:
                copies[1 - slot] = start(chunk + 1, 1 - slot)
            buf = bufs[slot]

            def body(i, V, buf=buf):
                v = buf[i]
                out = []
                for kk in range(K):
                    hi = jnp.maximum(V[kk], v)
                    v = jnp.minimum(V[kk], v)
                    out.append(hi)
                return tuple(out)

            V = lax.fori_loop(0, LC, body, V)

        for kk in range(K):
            outb[kk] = V[kk]
        pltpu.sync_copy(outb, out_hbm.at[b, pl.ds(0, K), pl.ds(c0, LANES)])


def kernel(inputs):
    return _topk_sc(inputs)
